# baseline (speedup 1.0000x reference)
import functools
import math

import jax
import jax.numpy as jnp
from jax import lax
from jax.experimental import pallas as pl
from jax.experimental.pallas import tpu as pltpu
from jax.experimental.pallas import tpu_sc as plsc

EMB = 64
SCALE = math.sqrt(EMB)
GSZ = 128
PITCH = 65


def _make_sc_kernel(batch, hist, V):
    info = plsc.get_sparse_core_info()
    NC, NS = info.num_cores, info.num_subcores
    NW = NC * NS
    n_btile = batch // GSZ
    n_pairs = hist * n_btile
    per_w = n_pairs // NW
    assert n_pairs % (2 * NW) == 0 and batch % GSZ == 0 and EMB % 16 == 0
    n_kk = per_w // 2

    mesh = plsc.VectorSubcoreMesh(core_axis_name="c", subcore_axis_name="s")

    @functools.partial(
        pl.kernel,
        mesh=mesh,
        compiler_params=pltpu.CompilerParams(
            use_tc_tiling_on_sc=False, needs_layout_passes=False
        ),
        out_type=jax.ShapeDtypeStruct(
            (hist, EMB // 8, n_btile, 8, GSZ), jnp.float32
        ),
        scratch_types=[
            pltpu.VMEM((per_w, GSZ), jnp.int32),
            pltpu.VMEM((GSZ, EMB), jnp.float32),
            pltpu.VMEM((GSZ, EMB), jnp.float32),
            pltpu.VMEM((GSZ, PITCH), jnp.float32),
            pltpu.VMEM((EMB // 8, 8, GSZ), jnp.float32),
            pltpu.VMEM((EMB // 8, 8, GSZ), jnp.float32),
            pltpu.SemaphoreType.DMA,
            pltpu.SemaphoreType.DMA,
            pltpu.SemaphoreType.DMA,
            pltpu.SemaphoreType.DMA,
        ],
    )
    def k(x_hbm, table_hbm, out_hbm, idx_v, rows0, rows1, stage,
          ob0, ob1, g0, g1, s0, s1):
        wid = lax.axis_index("s") * NC + lax.axis_index("c")
        pltpu.sync_copy(x_hbm.at[wid], idx_v)

        rows = (rows0, rows1)
        obuf = (ob0, ob1)
        gsem = (g0, g1)
        ssem = (s0, s1)
        gdummy = out_hbm.at[0, :, 0]

        def fire(t, b):
            pltpu.async_copy(
                table_hbm.at[idx_v.at[t]], rows[b], gsem[b]
            )

        lanes16 = lax.iota(jnp.int32, 16)

        def stage_scale(b):
            def srow(i, _):
                for j in range(EMB // 16):
                    stage[i, pl.ds(j * 16, 16)] = (
                        rows[b][i, pl.ds(j * 16, 16)] * SCALE
                    )
                return 0

            lax.fori_loop(0, GSZ, srow, 0, unroll=4)

        def transpose_out(b):
            def scol(e, _):
                col = lax.broadcast(e, (16,))
                for m in range(GSZ // 16):
                    v = plsc.load_gather(
                        stage, [lanes16 + (16 * m), col]
                    )
                    obuf[b][e // 8, e % 8, pl.ds(16 * m, 16)] = v
                return 0

            lax.fori_loop(0, EMB, scol, 0, unroll=2)

        fire(0, 0)

        def body(kk, _):
            for b in range(2):
                cur = 2 * kk + b
                o = 1 - b
                if b == 0:
                    fire(cur + 1, o)
                else:

                    @pl.when(kk < n_kk - 1)
                    def _():
                        fire(cur + 1, o)

                pltpu.make_async_copy(
                    table_hbm.at[pl.ds(0, GSZ)], rows[b], gsem[b]
                ).wait()
                stage_scale(b)

                @pl.when(kk >= 1)
                def _():
                    pltpu.make_async_copy(obuf[b], gdummy, ssem[b]).wait()

                transpose_out(b)
                q = per_w * wid + cur
                h = 8 * (q // 256) + lax.rem(q, 8)
                bb = lax.rem(q, 256) // 8
                pltpu.async_copy(obuf[b], out_hbm.at[h, :, bb], ssem[b])
            return 0

        lax.fori_loop(0, n_kk, body, 0)
        pltpu.make_async_copy(ob0, gdummy, ssem[0]).wait()
        pltpu.make_async_copy(ob1, gdummy, ssem[1]).wait()

    def run(x, table):
        x4 = x.reshape(batch // GSZ, GSZ, hist // 8, 8)
        x4 = jnp.transpose(x4, (2, 0, 3, 1))
        xt = x4.reshape(NW, per_w, GSZ)
        o5 = k(xt, table)
        return o5

    return run


def kernel(x, table):
    batch, hist = x.shape
    o5 = _make_sc_kernel(batch, hist, table.shape[0])(
        x.astype(jnp.int32), table
    )
    return jnp.transpose(o5, (2, 4, 0, 1, 3)).reshape(batch, hist, EMB)


# speedup vs baseline: 2.0309x; 2.0309x over previous
import functools
import math

import jax
import jax.numpy as jnp
from jax import lax
from jax.experimental import pallas as pl
from jax.experimental.pallas import tpu as pltpu
from jax.experimental.pallas import tpu_sc as plsc

EMB = 64
SCALE = math.sqrt(EMB)
GSZ = 128
PITCH = 129


def _make_sc_kernel(batch, hist, V):
    info = plsc.get_sparse_core_info()
    NC, NS = info.num_cores, info.num_subcores
    NW = NC * NS
    n_btile = batch // GSZ
    n_pairs = hist * n_btile
    per_w = n_pairs // NW
    assert n_pairs % (2 * NW) == 0 and batch % GSZ == 0 and EMB % 16 == 0
    n_kk = per_w // 2

    mesh = plsc.VectorSubcoreMesh(core_axis_name="c", subcore_axis_name="s")

    @functools.partial(
        pl.kernel,
        mesh=mesh,
        compiler_params=pltpu.CompilerParams(
            use_tc_tiling_on_sc=False, needs_layout_passes=False
        ),
        out_type=jax.ShapeDtypeStruct(
            (hist, EMB // 8, n_btile, 8, GSZ), jnp.float32
        ),
        scratch_types=[
            pltpu.VMEM((per_w, GSZ), jnp.int32),
            pltpu.VMEM((GSZ, EMB), jnp.float32),
            pltpu.VMEM((GSZ, EMB), jnp.float32),
            pltpu.VMEM((EMB // 8, 8, PITCH), jnp.float32),
            pltpu.VMEM((EMB // 8, 8, PITCH), jnp.float32),
            pltpu.SemaphoreType.DMA,
            pltpu.SemaphoreType.DMA,
            pltpu.SemaphoreType.DMA,
            pltpu.SemaphoreType.DMA,
        ],
    )
    def k(x_hbm, table_hbm, out_hbm, idx_v, rows0, rows1,
          ob0, ob1, g0, g1, s0, s1):
        wid = lax.axis_index("s") * NC + lax.axis_index("c")
        pltpu.sync_copy(x_hbm.at[wid], idx_v)

        rows = (rows0, rows1)
        obuf = (ob0, ob1)
        gsem = (g0, g1)
        ssem = (s0, s1)
        gdummy = out_hbm.at[0, :, 0]

        def fire(t, b):
            pltpu.async_copy(table_hbm.at[idx_v.at[t]], rows[b], gsem[b])

        iota16 = lax.iota(jnp.int32, 16)
        evecs = [
            (iota16 + 16 * j) // 8 for j in range(EMB // 16)
        ]
        svecs = [
            lax.rem(iota16 + 16 * j, 8) for j in range(EMB // 16)
        ]

        def scatter_pass(b):
            @plsc.parallel_loop(0, GSZ, 1, unroll=4)
            def _(l):
                colv = lax.broadcast(l, (16,))
                for j in range(EMB // 16):
                    v = rows[b][l, pl.ds(16 * j, 16)] * SCALE
                    plsc.store_scatter(obuf[b], [evecs[j], svecs[j], colv], v)

        fire(0, 0)

        def body(kk, _):
            for b in range(2):
                cur = 2 * kk + b
                o = 1 - b
                if b == 0:
                    fire(cur + 1, o)
                else:

                    @pl.when(kk < n_kk - 1)
                    def _():
                        fire(cur + 1, o)

                pltpu.make_async_copy(
                    table_hbm.at[pl.ds(0, GSZ)], rows[b], gsem[b]
                ).wait()

                @pl.when(kk >= 1)
                def _():
                    pltpu.make_async_copy(
                        obuf[b].at[:, :, pl.ds(0, GSZ)], gdummy, ssem[b]
                    ).wait()

                scatter_pass(b)
                q = per_w * wid + cur
                h = 8 * (q // 256) + lax.rem(q, 8)
                bb = lax.rem(q, 256) // 8
                pltpu.async_copy(
                    obuf[b].at[:, :, pl.ds(0, GSZ)],
                    out_hbm.at[h, :, bb],
                    ssem[b],
                )
            return 0

        lax.fori_loop(0, n_kk, body, 0)
        pltpu.make_async_copy(
            ob0.at[:, :, pl.ds(0, GSZ)], gdummy, ssem[0]
        ).wait()
        pltpu.make_async_copy(
            ob1.at[:, :, pl.ds(0, GSZ)], gdummy, ssem[1]
        ).wait()

    def run(x, table):
        x4 = x.reshape(batch // GSZ, GSZ, hist // 8, 8)
        x4 = jnp.transpose(x4, (2, 0, 3, 1))
        xt = x4.reshape(NW, per_w, GSZ)
        return k(xt, table)

    return run


def kernel(x, table):
    batch, hist = x.shape
    o5 = _make_sc_kernel(batch, hist, table.shape[0])(
        x.astype(jnp.int32), table
    )
    return jnp.transpose(o5, (2, 4, 0, 1, 3)).reshape(batch, hist, EMB)
